# Initial kernel scaffold; baseline (speedup 1.0000x reference)
import jax
import jax.numpy as jnp
from jax.experimental import pallas as pl


def kernel(node_emb, params, edge_index, edge_type):
    raise NotImplementedError("write your pallas kernel here")


# speedup vs baseline: 1.1024x; 1.1024x over previous
import functools
import math

import jax
import jax.numpy as jnp
from jax.experimental import pallas as pl
from jax.experimental.pallas import tpu as pltpu

N_NODES = 10000
E_EDGES = 320000
IN_DIM = 128
D = 128
N_ETYPE = 38
N_LAYER = 2
HEADS = 4
DPH = D // HEADS
NUM_TAGS = 9

_ROWS_BLK = 1000


def _gelu(x):
    return jax.nn.gelu(x, approximate=False)


def _mm_body(x_ref, w_ref, b_ref, o_ref, *, act):
    y = jnp.dot(x_ref[...], w_ref[...], preferred_element_type=jnp.float32)
    y = y + b_ref[...]
    if act == "relu":
        y = jnp.maximum(y, 0.0)
    o_ref[...] = y


def _mm(x, w, b, act="none"):
    n, k = x.shape
    m = w.shape[1]
    grid = (n // _ROWS_BLK,)
    return pl.pallas_call(
        functools.partial(_mm_body, act=act),
        grid=grid,
        in_specs=[
            pl.BlockSpec((_ROWS_BLK, k), lambda i: (i, 0)),
            pl.BlockSpec((k, m), lambda i: (0, 0)),
            pl.BlockSpec((1, m), lambda i: (0, 0)),
        ],
        out_specs=pl.BlockSpec((_ROWS_BLK, m), lambda i: (i, 0)),
        out_shape=jax.ShapeDtypeStruct((n, m), jnp.float32),
    )(x, w, b.reshape(1, m))


def _mm2_body(x_ref, y_ref, wx_ref, wy_ref, b_ref, o_ref):
    z = jnp.dot(x_ref[...], wx_ref[...], preferred_element_type=jnp.float32)
    z = z + jnp.dot(y_ref[...], wy_ref[...], preferred_element_type=jnp.float32)
    o_ref[...] = z + b_ref[...]


def _mm2(x, y, wx, wy, b):
    n, k = x.shape
    m = wx.shape[1]
    grid = (n // _ROWS_BLK,)
    return pl.pallas_call(
        _mm2_body,
        grid=grid,
        in_specs=[
            pl.BlockSpec((_ROWS_BLK, k), lambda i: (i, 0)),
            pl.BlockSpec((_ROWS_BLK, y.shape[1]), lambda i: (i, 0)),
            pl.BlockSpec((k, m), lambda i: (0, 0)),
            pl.BlockSpec((y.shape[1], m), lambda i: (0, 0)),
            pl.BlockSpec((1, m), lambda i: (0, 0)),
        ],
        out_specs=pl.BlockSpec((_ROWS_BLK, m), lambda i: (i, 0)),
        out_shape=jax.ShapeDtypeStruct((n, m), jnp.float32),
    )(x, y, wx, wy, b.reshape(1, m))


def _edge_phase(Qn, K1, M1, K2, M2, src, dst, et, deg):
    Et = src.shape[0]
    q = Qn[src].reshape(Et, HEADS, DPH)
    k = (K1[dst] + K2[et]).reshape(Et, HEADS, DPH)
    scores = (q * k).sum(axis=2) * (1.0 / math.sqrt(DPH))
    gmax = jnp.max(scores)
    e = jnp.exp(scores - gmax)
    dsum = jax.ops.segment_sum(e, src, num_segments=N_NODES)
    alpha = e / (dsum[src] + 1e-16) * deg[src][:, None]
    msg = (M1[src] + M2[et]).reshape(Et, HEADS, DPH)
    out_e = (msg * alpha[:, :, None]).reshape(Et, D)
    return jax.ops.segment_sum(out_e, dst, num_segments=N_NODES)


def kernel(node_emb, params, edge_index, edge_type):
    p = params
    N = N_NODES
    inv_bn = 1.0 / math.sqrt(1.0 + 1e-5)

    H = _gelu(_mm(node_emb, p['lm2gnn_w'], p['lm2gnn_b']))
    X = H

    src = jnp.concatenate([edge_index[0], jnp.arange(N, dtype=edge_index.dtype)])
    dst = jnp.concatenate([edge_index[1], jnp.arange(N, dtype=edge_index.dtype)])
    et = jnp.concatenate([edge_type, jnp.full((N,), N_ETYPE, edge_type.dtype)])
    Et = src.shape[0]
    deg = jax.ops.segment_sum(jnp.ones((Et,), jnp.float32), src, num_segments=N)

    for l in range(N_LAYER):
        h = p['edge_w1'][l] + p['edge_b1'][l]
        h = jnp.maximum(h * inv_bn * p['edge_g1'][l] + p['edge_be1'][l], 0.0)
        table = h @ p['edge_w2'][l] + p['edge_b2'][l]
        K2 = table @ p['k_w'][l][D:] + p['k_b'][l]
        M2 = table @ p['m_w'][l][D:] + p['m_b'][l]

        Wcat = jnp.concatenate([p['q_w'][l], p['k_w'][l][:D], p['m_w'][l][:D]], axis=1)
        bcat = jnp.concatenate([p['q_b'][l], jnp.zeros((2 * D,), jnp.float32)])
        QKM = _mm(X, Wcat, bcat)
        Qn, K1, M1 = QKM[:, :D], QKM[:, D:2 * D], QKM[:, 2 * D:]

        aggr = _edge_phase(Qn, K1, M1, K2, M2, src, dst, et, deg)

        g1 = _mm(aggr, p['mlp_w1'][l] * inv_bn * p['mlp_g'][l][None, :],
                 p['mlp_b1'][l] * inv_bn * p['mlp_g'][l] + p['mlp_be'][l], act="relu")
        X = _gelu(_mm(g1, p['mlp_w2'][l], p['mlp_b2'][l]))

    hidden = _gelu(_mm2(H, X, p['fo_w'], p['fc_w'], p['fo_b'] + p['fc_b']))
    tag_w = jnp.zeros((D, 128), jnp.float32).at[:, :NUM_TAGS].set(p['tag_w'])
    tag_b = jnp.zeros((128,), jnp.float32).at[:NUM_TAGS].set(p['tag_b'])
    em = _mm(hidden, tag_w, tag_b)[:, :NUM_TAGS]

    start, end, trans = p['crf_start'], p['crf_end'], p['crf_trans']

    def step(score, emit):
        total = score[:, None] + trans + emit[None, :]
        return jnp.max(total, axis=0), jnp.argmax(total, axis=0)

    init = start + em[0]
    final, hist = jax.lax.scan(step, init, em[1:])
    last = jnp.argmax(final + end)

    def back(tag, row):
        return row[tag], tag

    first, tags = jax.lax.scan(back, last, hist, reverse=True)
    return jnp.concatenate([first[None], tags])[None, :]


# speedup vs baseline: 3.1278x; 2.8372x over previous
import functools
import math

import jax
import jax.numpy as jnp
from jax.experimental import pallas as pl
from jax.experimental.pallas import tpu as pltpu

N_NODES = 10000
E_EDGES = 320000
IN_DIM = 128
D = 128
N_ETYPE = 38
N_LAYER = 2
HEADS = 4
DPH = D // HEADS
NUM_TAGS = 9

_ROWS_BLK = 1000


def _gelu(x):
    return jax.nn.gelu(x, approximate=False)


def _mm_body(x_ref, w_ref, b_ref, o_ref, *, act):
    y = jnp.dot(x_ref[...], w_ref[...], preferred_element_type=jnp.float32)
    y = y + b_ref[...]
    if act == "relu":
        y = jnp.maximum(y, 0.0)
    o_ref[...] = y


def _mm(x, w, b, act="none"):
    n, k = x.shape
    m = w.shape[1]
    grid = (n // _ROWS_BLK,)
    return pl.pallas_call(
        functools.partial(_mm_body, act=act),
        grid=grid,
        in_specs=[
            pl.BlockSpec((_ROWS_BLK, k), lambda i: (i, 0)),
            pl.BlockSpec((k, m), lambda i: (0, 0)),
            pl.BlockSpec((1, m), lambda i: (0, 0)),
        ],
        out_specs=pl.BlockSpec((_ROWS_BLK, m), lambda i: (i, 0)),
        out_shape=jax.ShapeDtypeStruct((n, m), jnp.float32),
    )(x, w, b.reshape(1, m))


def _mm2_body(x_ref, y_ref, wx_ref, wy_ref, b_ref, o_ref):
    z = jnp.dot(x_ref[...], wx_ref[...], preferred_element_type=jnp.float32)
    z = z + jnp.dot(y_ref[...], wy_ref[...], preferred_element_type=jnp.float32)
    o_ref[...] = z + b_ref[...]


def _mm2(x, y, wx, wy, b):
    n, k = x.shape
    m = wx.shape[1]
    grid = (n // _ROWS_BLK,)
    return pl.pallas_call(
        _mm2_body,
        grid=grid,
        in_specs=[
            pl.BlockSpec((_ROWS_BLK, k), lambda i: (i, 0)),
            pl.BlockSpec((_ROWS_BLK, y.shape[1]), lambda i: (i, 0)),
            pl.BlockSpec((k, m), lambda i: (0, 0)),
            pl.BlockSpec((y.shape[1], m), lambda i: (0, 0)),
            pl.BlockSpec((1, m), lambda i: (0, 0)),
        ],
        out_specs=pl.BlockSpec((_ROWS_BLK, m), lambda i: (i, 0)),
        out_shape=jax.ShapeDtypeStruct((n, m), jnp.float32),
    )(x, y, wx, wy, b.reshape(1, m))


_T_STEPS = N_NODES


def _viterbi_body(em_ref, s0_ref, transP_ref, endP_ref, out_ref, histA_ref):
    transP = transP_ref[...]
    sub_iota = jax.lax.broadcasted_iota(jnp.int32, (16, 128), 0)
    lane_iota = jax.lax.broadcasted_iota(jnp.int32, (1, 128), 1)
    big = jnp.int32(127)
    neg_rows = jnp.full((16 - NUM_TAGS, 128), jnp.float32(-1e30))

    def fwd(t, carry):
        s_colb, _ = carry
        em_b = jnp.broadcast_to(em_ref[pl.ds(t, 1), :], (16, 128))
        total = (s_colb + transP) + em_b
        srow = jnp.max(total, axis=0, keepdims=True)
        arg = jnp.min(jnp.where(total == srow, sub_iota, big), axis=0, keepdims=True)
        histA_ref[pl.ds(t - 1, 1), :] = arg
        rows = [jnp.broadcast_to(srow[:, i:i + 1], (1, 128)) for i in range(NUM_TAGS)]
        s_colb = jnp.concatenate(rows + [neg_rows], axis=0)
        return s_colb, srow

    s_colb = s0_ref[...]
    _, srow = jax.lax.fori_loop(1, _T_STEPS, fwd, (s_colb, s_colb[0:1, :]))

    final = srow + endP_ref[...]
    fm = jnp.max(final)
    last = jnp.min(jnp.where(final == fm, lane_iota, big))
    out_ref[0, _T_STEPS - 1] = last

    def bwd(ii, tag):
        t = _T_STEPS - 1 - ii
        row = histA_ref[pl.ds(t - 1, 1), :]
        prev = jnp.sum(jnp.where(lane_iota == tag, row, 0))
        out_ref[0, t - 1] = prev
        return prev

    jax.lax.fori_loop(0, _T_STEPS - 1, bwd, last)


def _viterbi_pallas(em_full, s0, transP, endP):
    return pl.pallas_call(
        _viterbi_body,
        in_specs=[
            pl.BlockSpec(memory_space=pltpu.VMEM),
            pl.BlockSpec(memory_space=pltpu.VMEM),
            pl.BlockSpec(memory_space=pltpu.VMEM),
            pl.BlockSpec(memory_space=pltpu.VMEM),
        ],
        out_specs=pl.BlockSpec(memory_space=pltpu.SMEM),
        out_shape=jax.ShapeDtypeStruct((1, _T_STEPS), jnp.int32),
        scratch_shapes=[pltpu.VMEM((_T_STEPS, 128), jnp.int32)],
    )(em_full, s0, transP, endP)


def _edge_phase(Qn, K1, M1, K2, M2, src, dst, et, deg):
    Et = src.shape[0]
    q = Qn[src].reshape(Et, HEADS, DPH)
    k = (K1[dst] + K2[et]).reshape(Et, HEADS, DPH)
    scores = (q * k).sum(axis=2) * (1.0 / math.sqrt(DPH))
    gmax = jnp.max(scores)
    e = jnp.exp(scores - gmax)
    dsum = jax.ops.segment_sum(e, src, num_segments=N_NODES)
    alpha = e / (dsum[src] + 1e-16) * deg[src][:, None]
    msg = (M1[src] + M2[et]).reshape(Et, HEADS, DPH)
    out_e = (msg * alpha[:, :, None]).reshape(Et, D)
    return jax.ops.segment_sum(out_e, dst, num_segments=N_NODES)


def kernel(node_emb, params, edge_index, edge_type):
    p = params
    N = N_NODES
    inv_bn = 1.0 / math.sqrt(1.0 + 1e-5)

    H = _gelu(_mm(node_emb, p['lm2gnn_w'], p['lm2gnn_b']))
    X = H

    src = jnp.concatenate([edge_index[0], jnp.arange(N, dtype=edge_index.dtype)])
    dst = jnp.concatenate([edge_index[1], jnp.arange(N, dtype=edge_index.dtype)])
    et = jnp.concatenate([edge_type, jnp.full((N,), N_ETYPE, edge_type.dtype)])
    Et = src.shape[0]
    deg = jax.ops.segment_sum(jnp.ones((Et,), jnp.float32), src, num_segments=N)

    for l in range(N_LAYER):
        h = p['edge_w1'][l] + p['edge_b1'][l]
        h = jnp.maximum(h * inv_bn * p['edge_g1'][l] + p['edge_be1'][l], 0.0)
        table = h @ p['edge_w2'][l] + p['edge_b2'][l]
        K2 = table @ p['k_w'][l][D:] + p['k_b'][l]
        M2 = table @ p['m_w'][l][D:] + p['m_b'][l]

        Wcat = jnp.concatenate([p['q_w'][l], p['k_w'][l][:D], p['m_w'][l][:D]], axis=1)
        bcat = jnp.concatenate([p['q_b'][l], jnp.zeros((2 * D,), jnp.float32)])
        QKM = _mm(X, Wcat, bcat)
        Qn, K1, M1 = QKM[:, :D], QKM[:, D:2 * D], QKM[:, 2 * D:]

        aggr = _edge_phase(Qn, K1, M1, K2, M2, src, dst, et, deg)

        g1 = _mm(aggr, p['mlp_w1'][l] * inv_bn * p['mlp_g'][l][None, :],
                 p['mlp_b1'][l] * inv_bn * p['mlp_g'][l] + p['mlp_be'][l], act="relu")
        X = _gelu(_mm(g1, p['mlp_w2'][l], p['mlp_b2'][l]))

    hidden = _gelu(_mm2(H, X, p['fo_w'], p['fc_w'], p['fo_b'] + p['fc_b']))
    tag_w = jnp.zeros((D, 128), jnp.float32).at[:, :NUM_TAGS].set(p['tag_w'])
    tag_b = jnp.zeros((128,), jnp.float32).at[:NUM_TAGS].set(p['tag_b'])
    em_full = _mm(hidden, tag_w, tag_b)

    start, end, trans = p['crf_start'], p['crf_end'], p['crf_trans']
    NEG = jnp.float32(-1e30)
    transP = jnp.full((16, 128), NEG).at[:NUM_TAGS, :NUM_TAGS].set(trans)
    endP = jnp.full((1, 128), NEG).at[0, :NUM_TAGS].set(end)
    s0 = jnp.full((16, 128), NEG)
    s0 = s0.at[:NUM_TAGS, :].set((start + em_full[0, :NUM_TAGS])[:, None])
    tags = _viterbi_pallas(em_full, s0, transP, endP)
    return tags
